# baseline (speedup 1.0000x reference)
import functools
import jax
import jax.numpy as jnp
import numpy as np
from jax import lax
from jax.experimental import pallas as pl
from jax.experimental.pallas import tpu as pltpu
from jax.experimental.pallas import tpu_sc as plsc

SQC = 0.1
CC = 0.01
EPS = 1e-7
KN = 20
KP = 32
B = 8
N = 1024
BN = B * N


def _lrelu(x):
    return jnp.where(x >= 0, x, 0.2 * x)


def _artanh(x):
    x = jnp.clip(x, -1.0 + 1e-5, 1.0 - 1e-5)
    return 0.5 * (jnp.log1p(x) - jnp.log1p(-x))


def _rownorm(x2sum):
    return jnp.sqrt(jnp.maximum(x2sum, EPS * EPS))


def _bf(v):
    return v.astype(jnp.bfloat16)


def _p1a_body(xt_ref, xh_ref, xx_ref):
    xt = xt_ref[0]
    xx = ((xt[:, 0:1] * xt[:, 0:1] + xt[:, 1:2] * xt[:, 1:2])
          + xt[:, 2:3] * xt[:, 2:3])
    xx_ref[0] = xx
    n = _rownorm(xx)
    s = jnp.minimum(jnp.tanh(SQC * n), 0.999) / (SQC * n)
    xh8 = xt * s
    xh_ref[0] = jnp.concatenate([xh8, jnp.zeros_like(xh8)], axis=1)


def _p1a(xt8):
    return pl.pallas_call(
        _p1a_body,
        grid=(B,),
        in_specs=[pl.BlockSpec((1, N, 8), lambda b: (b, 0, 0))],
        out_specs=[
            pl.BlockSpec((1, N, 16), lambda b: (b, 0, 0)),
            pl.BlockSpec((1, N, 1), lambda b: (b, 0, 0)),
        ],
        out_shape=[
            jax.ShapeDtypeStruct((B, N, 16), jnp.float32),
            jax.ShapeDtypeStruct((B, N, 1), jnp.float32),
        ],
    )(xt8)


def _p1b_body(xt_ref, xr_ref, xx_ref, xxr_ref, pd_ref):
    xt = _bf(xt_ref[0]).astype(jnp.float32)
    xr = _bf(xr_ref[0]).astype(jnp.float32)
    g = xt[:, 0:1] * xr[0:1, :]
    g = g + xt[:, 1:2] * xr[1:2, :]
    g = g + xt[:, 2:3] * xr[2:3, :]
    inner = -2.0 * g
    pd_ref[0] = (-xx_ref[0] - inner) - xxr_ref[0]


def _p1b(xt8, xrow, xx, xxr):
    return pl.pallas_call(
        _p1b_body,
        grid=(B,),
        in_specs=[
            pl.BlockSpec((1, N, 8), lambda b: (b, 0, 0)),
            pl.BlockSpec((1, 8, N), lambda b: (b, 0, 0)),
            pl.BlockSpec((1, N, 1), lambda b: (b, 0, 0)),
            pl.BlockSpec((1, 1, N), lambda b: (b, 0, 0)),
        ],
        out_specs=pl.BlockSpec((1, N, N), lambda b: (b, 0, 0)),
        out_shape=jax.ShapeDtypeStruct((B, N, N), jnp.float32),
    )(xt8, xrow, xx, xxr)


def _pd_body(f_ref, xx_ref, xxr_ref, pd_ref):
    xb = _bf(f_ref[0])
    g = lax.dot_general(xb, xb, (((1,), (1,)), ((), ())),
                        preferred_element_type=jnp.float32)
    inner = -2.0 * g
    pd_ref[0] = (-xx_ref[0] - inner) - xxr_ref[0]


def _pd(f, xx, xxr):
    c = f.shape[-1]
    return pl.pallas_call(
        _pd_body,
        grid=(B,),
        in_specs=[
            pl.BlockSpec((1, N, c), lambda b: (b, 0, 0)),
            pl.BlockSpec((1, N, 1), lambda b: (b, 0, 0)),
            pl.BlockSpec((1, 1, N), lambda b: (b, 0, 0)),
        ],
        out_specs=pl.BlockSpec((1, N, N), lambda b: (b, 0, 0)),
        out_shape=jax.ShapeDtypeStruct((B, N, N), jnp.float32),
    )(f, xx, xxr)


def _p2_body(feat_ref, xh_ref, w1_ref, u_ref, st_ref, acc_ref):
    i = pl.program_id(0) * pl.num_programs(1) + pl.program_id(1)
    a = feat_ref[0, 0]
    r = xh_ref[0]

    x2 = jnp.sum(a * a, axis=1, keepdims=True)
    y2 = jnp.sum(r * r, axis=1, keepdims=True)
    xy = -jnp.sum(a * r, axis=1, keepdims=True)
    num = (1.0 + 2.0 * CC * xy + CC * y2) * a - (1.0 - CC * x2) * r
    den = 1.0 + 2.0 * CC * xy + CC * CC * x2 * y2
    ma = num / jnp.maximum(den, EPS)

    xn = _rownorm(jnp.sum(ma * ma, axis=1, keepdims=True) + y2)
    e6 = jnp.concatenate([ma[:, 0:3], r[:, 0:3], jnp.zeros((N, 2), jnp.float32)],
                         axis=1)
    mx = jnp.dot(_bf(e6), _bf(w1_ref[...]), preferred_element_type=jnp.float32)
    mxn = _rownorm(jnp.sum(mx * mx, axis=1, keepdims=True))
    arg = mxn / xn * _artanh(SQC * xn)
    th = jnp.minimum(jnp.tanh(arg), 0.999)
    hn = jnp.maximum(th / SQC, EPS)
    s_u = (_artanh(SQC * hn) / (SQC * hn)) * (th / (SQC * mxn))
    u = mx * s_u
    u_ref[0, 0] = u

    @pl.when(i == 0)
    def _init():
        acc_ref[...] = jnp.zeros_like(acc_ref)

    acc_ref[0:1, :] += jnp.sum(u, axis=0, keepdims=True)
    acc_ref[1:2, :] += jnp.sum(u * u, axis=0, keepdims=True)

    @pl.when(i == B * KN - 1)
    def _fin():
        st_ref[...] = acc_ref[...]


def _p2(feat, xh16, w1p):
    return pl.pallas_call(
        _p2_body,
        grid=(B, KN),
        in_specs=[
            pl.BlockSpec((1, 1, N, 16), lambda b, k: (b, k, 0, 0)),
            pl.BlockSpec((1, N, 16), lambda b, k: (b, 0, 0)),
            pl.BlockSpec((8, 64), lambda b, k: (0, 0)),
        ],
        out_specs=[
            pl.BlockSpec((1, 1, N, 64), lambda b, k: (b, k, 0, 0)),
            pl.BlockSpec((2, 64), lambda b, k: (0, 0)),
        ],
        out_shape=[
            jax.ShapeDtypeStruct((B, KN, N, 64), jnp.float32),
            jax.ShapeDtypeStruct((2, 64), jnp.float32),
        ],
        scratch_shapes=[pltpu.VMEM((2, 64), jnp.float32)],
    )(feat, xh16, w1p)


def _p3_body(u_ref, st_ref, x1_ref, xx_ref, m_ref):
    k = pl.program_id(1)
    cnt = float(B * N * KN)
    mean = st_ref[0:1, :] / cnt
    var = st_ref[1:2, :] / cnt - mean * mean
    inv = lax.rsqrt(var + 1e-5)

    @pl.when(k == 0)
    def _init():
        m_ref[...] = jnp.full_like(m_ref, -jnp.inf)

    u = u_ref[0, 0]
    uh = (u - mean) * inv
    nu = _rownorm(jnp.sum(uh * uh, axis=1, keepdims=True))
    s1 = jnp.minimum(jnp.tanh(SQC * nu), 0.999) / (SQC * nu)
    w = uh * s1
    nw = jnp.maximum(jnp.minimum(jnp.tanh(SQC * nu), 0.999) / SQC, EPS)
    s2 = _artanh(SQC * nw) / (SQC * nw)
    v = _lrelu(w * s2)
    nv = _rownorm(jnp.sum(v * v, axis=1, keepdims=True))
    s3 = jnp.minimum(jnp.tanh(SQC * nv), 0.999) / (SQC * nv)
    m_ref[...] = jnp.maximum(m_ref[...], v * s3)

    @pl.when(k == KN - 1)
    def _tail():
        x1m = m_ref[...]
        n1 = _rownorm(jnp.sum(x1m * x1m, axis=1, keepdims=True))
        x1 = _artanh(SQC * n1) * x1m / (SQC * n1)
        x1_ref[0] = x1
        xx_ref[0] = jnp.sum(x1 * x1, axis=1, keepdims=True)


def _p3(u, st):
    return pl.pallas_call(
        _p3_body,
        grid=(B, KN),
        in_specs=[
            pl.BlockSpec((1, 1, N, 64), lambda b, k: (b, k, 0, 0)),
            pl.BlockSpec((2, 64), lambda b, k: (0, 0)),
        ],
        out_specs=[
            pl.BlockSpec((1, N, 64), lambda b, k: (b, 0, 0)),
            pl.BlockSpec((1, N, 1), lambda b, k: (b, 0, 0)),
        ],
        out_shape=[
            jax.ShapeDtypeStruct((B, N, 64), jnp.float32),
            jax.ShapeDtypeStruct((B, N, 1), jnp.float32),
        ],
        scratch_shapes=[pltpu.VMEM((N, 64), jnp.float32)],
    )(u, st)


def _c1_body(feat_ref, x_ref, wl_ref, wr_ref, hm_ref, st_ref,
             m_ref, acc_ref):
    b = pl.program_id(0)
    k = pl.program_id(1)
    xc = x_ref[0]
    d = feat_ref[0, 0] - xc
    h = (jnp.dot(_bf(d), _bf(wl_ref[...]), preferred_element_type=jnp.float32)
         + jnp.dot(_bf(xc), _bf(wr_ref[...]), preferred_element_type=jnp.float32))

    i = b * pl.num_programs(1) + k

    @pl.when(i == 0)
    def _init0():
        acc_ref[...] = jnp.zeros_like(acc_ref)

    @pl.when(k == 0)
    def _initb():
        m_ref[...] = jnp.full_like(m_ref, -jnp.inf)

    acc_ref[0:1, :] += jnp.sum(h, axis=0, keepdims=True)
    acc_ref[1:2, :] += jnp.sum(h * h, axis=0, keepdims=True)
    m_ref[...] = jnp.maximum(m_ref[...], h)

    @pl.when(k == KN - 1)
    def _wr():
        hm_ref[0] = m_ref[...]

    @pl.when(i == B * KN - 1)
    def _fin():
        cnt = float(B * N * KN)
        mean = acc_ref[0:1, :] / cnt
        var = acc_ref[1:2, :] / cnt - mean * mean
        st_ref[0:1, :] = mean
        st_ref[1:2, :] = lax.rsqrt(var + 1e-5)


def _c1(feat, xs, wl, wr):
    c = xs.shape[-1]
    o = wl.shape[1]
    return pl.pallas_call(
        _c1_body,
        grid=(B, KN),
        in_specs=[
            pl.BlockSpec((1, 1, N, c), lambda b, k: (b, k, 0, 0)),
            pl.BlockSpec((1, N, c), lambda b, k: (b, 0, 0)),
            pl.BlockSpec((c, o), lambda b, k: (0, 0)),
            pl.BlockSpec((c, o), lambda b, k: (0, 0)),
        ],
        out_specs=[
            pl.BlockSpec((1, N, o), lambda b, k: (b, 0, 0)),
            pl.BlockSpec((2, o), lambda b, k: (0, 0)),
        ],
        out_shape=[
            jax.ShapeDtypeStruct((B, N, o), jnp.float32),
            jax.ShapeDtypeStruct((2, o), jnp.float32),
        ],
        scratch_shapes=[pltpu.VMEM((N, o), jnp.float32),
                        pltpu.VMEM((2, o), jnp.float32)],
    )(feat, xs, wl, wr)


def _c2_body(hm_ref, st_ref, xs_ref, xx_ref):
    xs = _lrelu((hm_ref[0] - st_ref[0:1, :]) * st_ref[1:2, :])
    xs_ref[0] = xs
    xx_ref[0] = jnp.sum(xs * xs, axis=1, keepdims=True)


def _c2(hm, st):
    o = hm.shape[-1]
    return pl.pallas_call(
        _c2_body,
        grid=(B,),
        in_specs=[
            pl.BlockSpec((1, N, o), lambda b: (b, 0, 0)),
            pl.BlockSpec((2, o), lambda b: (0, 0)),
        ],
        out_specs=[
            pl.BlockSpec((1, N, o), lambda b: (b, 0, 0)),
            pl.BlockSpec((1, N, 1), lambda b: (b, 0, 0)),
        ],
        out_shape=[
            jax.ShapeDtypeStruct((B, N, o), jnp.float32),
            jax.ShapeDtypeStruct((B, N, 1), jnp.float32),
        ],
    )(hm, st)


def _p6_body(hm_ref, st_ref, x1_in, x2_in, x3_in, w5_ref,
             out_ref, facc_ref, hmx_ref):
    b = pl.program_id(0)
    x4 = _lrelu((hm_ref[0] - st_ref[0:1, :]) * st_ref[1:2, :])
    cat = jnp.concatenate([x1_in[0], x2_in[0], x3_in[0], x4], axis=1)
    h = jnp.dot(_bf(cat), _bf(w5_ref[...]), preferred_element_type=jnp.float32)

    @pl.when(b == 0)
    def _init():
        facc_ref[...] = jnp.zeros_like(facc_ref)

    facc_ref[0:1, :] += jnp.sum(h, axis=0, keepdims=True)
    facc_ref[1:2, :] += jnp.sum(h * h, axis=0, keepdims=True)
    hmx_ref[b, :] = jnp.max(h, axis=0)

    @pl.when(b == B - 1)
    def _fin():
        cnt = float(B * N)
        fmean = facc_ref[0:1, :] / cnt
        fvar = facc_ref[1:2, :] / cnt - fmean * fmean
        finv = lax.rsqrt(fvar + 1e-5)
        out_ref[...] = _lrelu((hmx_ref[...] - fmean) * finv)


def _p6(hm, st, x1, x2, x3, w5t):
    o = hm.shape[-1]
    bmap = lambda i: (i, 0, 0)
    cmap = lambda i: (0, 0)
    return pl.pallas_call(
        _p6_body,
        grid=(B,),
        in_specs=[
            pl.BlockSpec((1, N, o), bmap),
            pl.BlockSpec((2, o), cmap),
            pl.BlockSpec((1, N, 64), bmap),
            pl.BlockSpec((1, N, 64), bmap),
            pl.BlockSpec((1, N, 128), bmap),
            pl.BlockSpec((512, 1024), cmap),
        ],
        out_specs=pl.BlockSpec((B, 1024), lambda i: (0, 0)),
        out_shape=jax.ShapeDtypeStruct((B, 1024), jnp.float32),
        scratch_shapes=[
            pltpu.VMEM((2, 1024), jnp.float32),
            pltpu.VMEM((B, 1024), jnp.float32),
        ],
    )(hm, st, x1, x2, x3, w5t)


def _topk_gidx_jax(pd):
    _, idx = lax.top_k(pd, KN)
    gidx = idx + (jnp.arange(B, dtype=jnp.int32) * N)[:, None, None]
    pad = jnp.zeros((B, N, KP - KN), jnp.int32)
    return jnp.concatenate([gidx.astype(jnp.int32), pad], axis=-1).reshape(BN, KP)


def _gather_rows_jax(tbl, gidx):
    c = tbl.shape[-1]
    g = tbl.reshape(BN, c)[gidx[:, :KN]]
    return jnp.transpose(g.reshape(B, N, KN, c), (0, 2, 1, 3))


_topk_gidx = _topk_gidx_jax
_gather_rows = _gather_rows_jax


def kernel(x, W1, W2, W3, W4, W5):
    f32 = jnp.float32
    xt = jnp.transpose(x, (0, 2, 1)).astype(f32)
    xt8 = jnp.concatenate([xt, jnp.zeros((B, N, 5), f32)], axis=-1)
    xrow = jnp.concatenate([x.astype(f32), jnp.zeros((B, 5, N), f32)], axis=1)

    w1p = jnp.zeros((8, 64), f32).at[0:6, :].set(W1.T)

    def _split(W):
        o2 = W.shape[1] // 2
        return W[:, :o2].T.astype(f32), W[:, o2:].T.astype(f32)

    w2l, w2r = _split(W2)
    w3l, w3r = _split(W3)
    w4l, w4r = _split(W4)

    xh16, xx1 = _p1a(xt8)
    pd1 = _p1b(xt8, xrow, xx1, xx1.reshape(B, 1, N))
    gidx1 = _topk_gidx(pd1)
    feat1 = _gather_rows(xh16, gidx1)
    u, st1 = _p2(feat1, xh16, w1p)
    x1, xx2 = _p3(u, st1)

    pd2 = _pd(x1, xx2, xx2.reshape(B, 1, N))
    gidx2 = _topk_gidx(pd2)
    feat2 = _gather_rows(x1, gidx2)
    hm2, st2 = _c1(feat2, x1, w2l, w2r)
    x2, xx3 = _c2(hm2, st2)

    pd3 = _pd(x2, xx3, xx3.reshape(B, 1, N))
    gidx3 = _topk_gidx(pd3)
    feat3 = _gather_rows(x2, gidx3)
    hm3, st3 = _c1(feat3, x2, w3l, w3r)
    x3, xx4 = _c2(hm3, st3)

    pd4 = _pd(x3, xx4, xx4.reshape(B, 1, N))
    gidx4 = _topk_gidx(pd4)
    feat4 = _gather_rows(x3, gidx4)
    hm4, st4 = _c1(feat4, x3, w4l, w4r)
    out = _p6(hm4, st4, x1, x2, x3, W5.T.astype(f32))
    return out
